# baseline (speedup 1.0000x reference)
import functools

import jax
import jax.numpy as jnp
from jax import lax
from jax.experimental import pallas as pl
from jax.experimental.pallas import tpu as pltpu
from jax.experimental.pallas import tpu_sc as plsc

D_MODEL = 64
SEQ = 200
BATCH = 4096
N_ROWS = BATCH * SEQ
NUM_CORES = 2
NUM_SUBCORES = 16
NW = NUM_CORES * NUM_SUBCORES
ROWS_PER_W = N_ROWS // NW
CHUNK = 128
NCHUNK = ROWS_PER_W // CHUNK


def _gather_body(table_hbm, idx_hbm, out_hbm, idx_v, rows0, rows1, g0, g1, s0, s1):
    wid = lax.axis_index("s") * NUM_CORES + lax.axis_index("c")
    pltpu.sync_copy(idx_hbm.at[wid], idx_v)

    pltpu.async_copy(table_hbm.at[idx_v.at[0]], rows0, g0)

    def body(j, carry):
        even = (j % 2) == 0

        @pl.when(even)
        def _():
            pltpu.make_async_copy(table_hbm.at[idx_v.at[j]], rows0, g0).wait()

            @pl.when(j + 1 < NCHUNK)
            def _():
                @pl.when(j >= 1)
                def _():
                    pltpu.make_async_copy(rows1, out_hbm.at[wid, j - 1], s1).wait()
                pltpu.async_copy(table_hbm.at[idx_v.at[j + 1]], rows1, g1)

            pltpu.async_copy(rows0, out_hbm.at[wid, j], s0)

        @pl.when(jnp.logical_not(even))
        def _():
            pltpu.make_async_copy(table_hbm.at[idx_v.at[j]], rows1, g1).wait()

            @pl.when(j + 1 < NCHUNK)
            def _():
                pltpu.make_async_copy(rows0, out_hbm.at[wid, j - 1], s0).wait()
                pltpu.async_copy(table_hbm.at[idx_v.at[j + 1]], rows0, g0)

            pltpu.async_copy(rows1, out_hbm.at[wid, j], s1)

        return carry

    lax.fori_loop(0, NCHUNK, body, 0)
    pltpu.make_async_copy(rows0, out_hbm.at[wid, NCHUNK - 2], s0).wait()
    pltpu.make_async_copy(rows1, out_hbm.at[wid, NCHUNK - 1], s1).wait()


@jax.jit
def _run(table, idx3):
    mesh = plsc.VectorSubcoreMesh(core_axis_name="c", subcore_axis_name="s")
    f = pl.kernel(
        _gather_body,
        out_type=jax.ShapeDtypeStruct((NW, NCHUNK, CHUNK, D_MODEL), jnp.float32),
        mesh=mesh,
        compiler_params=pltpu.CompilerParams(use_tc_tiling_on_sc=False),
        scratch_types=[
            pltpu.VMEM((NCHUNK, CHUNK), jnp.int32),
            pltpu.VMEM((CHUNK, D_MODEL), jnp.float32),
            pltpu.VMEM((CHUNK, D_MODEL), jnp.float32),
            pltpu.SemaphoreType.DMA,
            pltpu.SemaphoreType.DMA,
            pltpu.SemaphoreType.DMA,
            pltpu.SemaphoreType.DMA,
        ],
    )
    return f(table, idx3)


def kernel(time_ids, pe_weight):
    idx3 = time_ids.reshape(NW, NCHUNK, CHUNK).astype(jnp.int32)
    out = _run(pe_weight, idx3)
    return out.reshape(BATCH, SEQ, D_MODEL)
